# baseline (speedup 1.0000x reference)
import functools

import jax
import jax.numpy as jnp
import numpy as np
from jax.experimental import pallas as pl
from jax.experimental.pallas import tpu as pltpu

_RC = 6.0
_BLOCK_I = 256


def _pair_kernel(pos_blk_ref, pos_t_ref, cel_ref,
                 adj_ref, sft_ref, vec_ref, sod_ref,
                 *, n, block_i, rc2):
    pid = pl.program_id(0)
    rows = pid * block_i + jax.lax.broadcasted_iota(jnp.int32, (block_i, n), 0)
    cols = jax.lax.broadcasted_iota(jnp.int32, (block_i, n), 1)

    sod = jnp.zeros((block_i, n), jnp.float32)
    vs = []
    for c in range(3):
        pj = pos_t_ref[c:c + 1, :]
        pi = pos_blk_ref[:, c:c + 1]
        dc = cel_ref[c, c]
        d = pj - pi
        f = d * (1.0 / dc)
        s = -jnp.round(f)
        v = d + s * dc
        sod = sod + v * v
        vs.append((v, s))

    mask = (sod < rc2) & (rows != cols)
    adj_ref[...] = mask.astype(jnp.int32)
    sod_ref[...] = jnp.where(mask, sod, 0.0)
    for c, (v, s) in enumerate(vs):
        vec_ref[c, :, :] = jnp.where(mask, v, 0.0)
        sft_ref[c, :, :] = jnp.where(mask, s, 0.0).astype(jnp.int32)


def kernel(pos, cel):
    n = pos.shape[0]
    block_i = _BLOCK_I
    grid = n // block_i
    rc2 = np.float32(_RC * _RC)

    pos_t = pos.T

    kfn = functools.partial(_pair_kernel, n=n, block_i=block_i, rc2=rc2)
    adj, sft_p, vec_p, sod = pl.pallas_call(
        kfn,
        grid=(grid,),
        compiler_params=pltpu.CompilerParams(vmem_limit_bytes=100 * 1024 * 1024),
        in_specs=[
            pl.BlockSpec((block_i, 3), lambda i: (i, 0)),
            pl.BlockSpec((3, n), lambda i: (0, 0)),
            pl.BlockSpec((3, 3), lambda i: (0, 0)),
        ],
        out_specs=[
            pl.BlockSpec((block_i, n), lambda i: (i, 0)),
            pl.BlockSpec((3, block_i, n), lambda i: (0, i, 0)),
            pl.BlockSpec((3, block_i, n), lambda i: (0, i, 0)),
            pl.BlockSpec((block_i, n), lambda i: (i, 0)),
        ],
        out_shape=[
            jax.ShapeDtypeStruct((n, n), jnp.int32),
            jax.ShapeDtypeStruct((3, n, n), jnp.int32),
            jax.ShapeDtypeStruct((3, n, n), jnp.float32),
            jax.ShapeDtypeStruct((n, n), jnp.float32),
        ],
    )(pos, pos_t, cel)

    return (adj, jnp.transpose(sft_p, (1, 2, 0)),
            jnp.transpose(vec_p, (1, 2, 0)), sod)


# speedup vs baseline: 1.0649x; 1.0649x over previous
import functools

import jax
import jax.numpy as jnp
import numpy as np
from jax.experimental import pallas as pl
from jax.experimental.pallas import tpu as pltpu

_RC = 6.0
_BLOCK_I = 128


def _pair_kernel(pos_blk_ref, pos_t_ref, cel_ref,
                 adj_ref, sft_ref, vec_ref, sod_ref,
                 *, n, block_i, rc2):
    pid = pl.program_id(0)
    rows = pid * block_i + jax.lax.broadcasted_iota(jnp.int32, (block_i, n), 0)
    cols = jax.lax.broadcasted_iota(jnp.int32, (block_i, n), 1)

    sod = jnp.zeros((block_i, n), jnp.float32)
    vs = []
    for c in range(3):
        pj = pos_t_ref[c:c + 1, :]
        pi = pos_blk_ref[:, c:c + 1]
        dc = cel_ref[c, c]
        d = pj - pi
        f = d * (1.0 / dc)
        s = -jnp.round(f)
        v = d + s * dc
        sod = sod + v * v
        vs.append((v, s))

    mask = (sod < rc2) & (rows != cols)
    adj_ref[...] = mask.astype(jnp.int32)
    sod_ref[...] = jnp.where(mask, sod, 0.0)
    for c, (v, s) in enumerate(vs):
        vec_ref[c, :, :] = jnp.where(mask, v, 0.0)
        sft_ref[c, :, :] = jnp.where(mask, s, 0.0).astype(jnp.int32)


def kernel(pos, cel):
    n = pos.shape[0]
    block_i = _BLOCK_I
    grid = n // block_i
    rc2 = np.float32(_RC * _RC)

    pos_t = pos.T

    kfn = functools.partial(_pair_kernel, n=n, block_i=block_i, rc2=rc2)
    adj, sft_p, vec_p, sod = pl.pallas_call(
        kfn,
        grid=(grid,),
        compiler_params=pltpu.CompilerParams(vmem_limit_bytes=100 * 1024 * 1024),
        in_specs=[
            pl.BlockSpec((block_i, 3), lambda i: (i, 0)),
            pl.BlockSpec((3, n), lambda i: (0, 0)),
            pl.BlockSpec((3, 3), lambda i: (0, 0)),
        ],
        out_specs=[
            pl.BlockSpec((block_i, n), lambda i: (i, 0)),
            pl.BlockSpec((3, block_i, n), lambda i: (0, i, 0)),
            pl.BlockSpec((3, block_i, n), lambda i: (0, i, 0)),
            pl.BlockSpec((block_i, n), lambda i: (i, 0)),
        ],
        out_shape=[
            jax.ShapeDtypeStruct((n, n), jnp.int32),
            jax.ShapeDtypeStruct((3, n, n), jnp.int32),
            jax.ShapeDtypeStruct((3, n, n), jnp.float32),
            jax.ShapeDtypeStruct((n, n), jnp.float32),
        ],
    )(pos, pos_t, cel)

    return (adj, jnp.transpose(sft_p, (1, 2, 0)),
            jnp.transpose(vec_p, (1, 2, 0)), sod)
